# Initial kernel scaffold; baseline (speedup 1.0000x reference)
import jax
import jax.numpy as jnp
from jax.experimental import pallas as pl


def kernel(ans_emb, ocr_emb, prev_inds, ans_w, ans_b, ocr_w, ocr_b, emb_w, emb_b, tt_table):
    raise NotImplementedError("write your pallas kernel here")


# speedup vs baseline: 3.0637x; 3.0637x over previous
import functools

import jax
import jax.numpy as jnp
from jax import lax
from jax.experimental import pallas as pl
from jax.experimental.pallas import tpu as pltpu
from jax.experimental.pallas import tpu_sc as plsc

HID = 768
NCHUNK = HID // 16
ANS_NUM = 5000
OCR_NUM = 50
BATCH = 32
DEC_LEN = 100
PAD_LEN = 128
ROWS_PER_GATHER = 32


def _rsqrt(x):
    xi = lax.bitcast_convert_type(x, jnp.int32)
    yi = jnp.int32(0x5F3759DF) - lax.shift_right_arithmetic(xi, 1)
    y = lax.bitcast_convert_type(yi, jnp.float32)
    for _ in range(3):
        y = y * (1.5 - 0.5 * x * y * y)
    return y


_GATHER_DNUMS = lax.GatherDimensionNumbers(
    offset_dims=(), collapsed_slice_dims=(0,), start_index_map=(0,))


def _permute(v, idx):
    return lax.gather(v, idx[:, None], _GATHER_DNUMS, slice_sizes=(1,),
                      mode=lax.GatherScatterMode.PROMISE_IN_BOUNDS)


def _lane_total(v):
    i = lax.iota(jnp.int32, 16)
    for st in (1, 2, 4, 8):
        v = v + _permute(v, i ^ st)
    return v


def _row_stats(read):
    def body(j, c):
        s, s2 = c
        x = read(j)
        return s + x, s2 + x * x
    zero = jnp.zeros((16,), jnp.float32)
    s, s2 = lax.fori_loop(0, NCHUNK, body, (zero, zero))
    mu = _lane_total(s) * (1.0 / HID)
    var = _lane_total(s2) * (1.0 / HID) - mu * mu
    return mu, _rsqrt(var + 1e-12)


def _sc_body(ans_hbm, ocr_hbm, prev_hbm, tt_hbm,
             ans_w_hbm, ans_b_hbm, ocr_w_hbm, ocr_b_hbm,
             emb_w_hbm, emb_b_hbm, out_hbm,
             idx_v, aidx_v, oidx_v, ans_rows, ocr_rows, stage,
             tt_v, pw0, pdw, pb0, pdb, ew, eb, sem):
    nc = 2
    wid = lax.axis_index("s") * nc + lax.axis_index("c")

    pltpu.sync_copy(prev_hbm.at[pl.ds(wid * PAD_LEN, PAD_LEN)], idx_v)

    ocr_base = wid * OCR_NUM
    for k in range(PAD_LEN // 16):
        v = idx_v[pl.ds(k * 16, 16)]
        t = v >= ANS_NUM
        aidx_v[pl.ds(k * 16, 16)] = jnp.where(t, 0, v)
        oidx_v[pl.ds(k * 16, 16)] = jnp.where(t, v - ANS_NUM + ocr_base, ocr_base)

    pltpu.sync_copy(ans_w_hbm, pw0)
    pltpu.sync_copy(ocr_w_hbm, pdw)
    pltpu.sync_copy(ans_b_hbm, pb0)
    pltpu.sync_copy(ocr_b_hbm, pdb)
    pltpu.sync_copy(emb_w_hbm, ew)
    pltpu.sync_copy(emb_b_hbm, eb)
    pltpu.sync_copy(tt_hbm.at[pl.ds(0, 2 * HID)], tt_v)

    mu0, rs0 = _row_stats(lambda j: tt_v[pl.ds(j * 16, 16)])
    mu1, rs1 = _row_stats(lambda j: tt_v[pl.ds(HID + j * 16, 16)])
    for j in range(NCHUNK):
        sl = pl.ds(j * 16, 16)
        pdw[sl] = pdw[sl] - pw0[sl]
        tte0 = (tt_v[sl] - mu0) * rs0 * ew[sl] + eb[sl]
        pb0[sl] = pb0[sl] + tte0
        tte1 = (tt_v[pl.ds(HID + j * 16, 16)] - mu1) * rs1 * ew[sl] + eb[sl]
        pdb[sl] = pdb[sl] + tte1 - pb0[sl]

    out_base = wid * DEC_LEN
    for c in range(PAD_LEN // ROWS_PER_GATHER):
        pltpu.async_copy(
            ans_hbm.at[aidx_v.at[pl.ds(c * ROWS_PER_GATHER, ROWS_PER_GATHER)]],
            ans_rows, sem).wait()
        pltpu.async_copy(
            ocr_hbm.at[oidx_v.at[pl.ds(c * ROWS_PER_GATHER, ROWS_PER_GATHER)]],
            ocr_rows, sem).wait()

        def row_body(r, carry, c=c):
            rg = c * ROWS_PER_GATHER + r
            idx_splat = plsc.load_gather(idx_v, [jnp.broadcast_to(rg, (16,))])
            t = idx_splat >= ANS_NUM
            tf = jnp.where(t, 1.0, 0.0).astype(jnp.float32)

            def read_x(j):
                xa = ans_rows[r, pl.ds(j * 16, 16)]
                xo = ocr_rows[r, pl.ds(j * 16, 16)]
                return jnp.where(t, xo, xa)

            mu_v, rs_v = _row_stats(read_x)

            def norm_body(j, carry2):
                sl = pl.ds(j * 16, 16)
                x = read_x(j)
                ws = pw0[sl] + tf * pdw[sl]
                a = ws * rs_v
                cc = (pb0[sl] + tf * pdb[sl]) - mu_v * a
                stage[pl.ds(r * HID + j * 16, 16)] = x * a + cc
                return carry2

            lax.fori_loop(0, NCHUNK, norm_body, 0)
            return carry

        nrows = min(ROWS_PER_GATHER, DEC_LEN - c * ROWS_PER_GATHER)
        if nrows <= 0:
            continue
        lax.fori_loop(0, nrows, row_body, 0)
        pltpu.sync_copy(
            stage.at[pl.ds(0, nrows * HID)],
            out_hbm.at[pl.ds((out_base + c * ROWS_PER_GATHER) * HID, nrows * HID)])


def kernel(ans_emb, ocr_emb, prev_inds, ans_w, ans_b, ocr_w, ocr_b, emb_w, emb_b, tt_table):
    batch = ocr_emb.shape[0]
    ocr_flat = ocr_emb.reshape(batch * OCR_NUM, HID)
    prev_pad = jnp.pad(prev_inds, ((0, 0), (0, PAD_LEN - DEC_LEN))).reshape(-1)
    tt_flat = tt_table.reshape(-1)

    mesh = plsc.VectorSubcoreMesh(core_axis_name="c", subcore_axis_name="s")
    run = functools.partial(
        pl.kernel,
        mesh=mesh,
        compiler_params=pltpu.CompilerParams(needs_layout_passes=False),
        out_type=jax.ShapeDtypeStruct((batch * DEC_LEN * HID,), jnp.float32),
        scratch_types=[
            pltpu.VMEM((PAD_LEN,), jnp.int32),
            pltpu.VMEM((PAD_LEN,), jnp.int32),
            pltpu.VMEM((PAD_LEN,), jnp.int32),
            pltpu.VMEM((ROWS_PER_GATHER, HID), jnp.float32),
            pltpu.VMEM((ROWS_PER_GATHER, HID), jnp.float32),
            pltpu.VMEM((ROWS_PER_GATHER * HID,), jnp.float32),
            pltpu.VMEM((2 * HID,), jnp.float32),
            pltpu.VMEM((HID,), jnp.float32),
            pltpu.VMEM((HID,), jnp.float32),
            pltpu.VMEM((HID,), jnp.float32),
            pltpu.VMEM((HID,), jnp.float32),
            pltpu.VMEM((HID,), jnp.float32),
            pltpu.VMEM((HID,), jnp.float32),
            pltpu.SemaphoreType.DMA,
        ],
    )(_sc_body)
    out = run(ans_emb, ocr_flat, prev_pad, tt_flat,
              ans_w, ans_b, ocr_w, ocr_b, emb_w, emb_b)
    return out.reshape(batch, DEC_LEN, HID)


# speedup vs baseline: 4.1583x; 1.3573x over previous
import functools

import jax
import jax.numpy as jnp
from jax import lax
from jax.experimental import pallas as pl
from jax.experimental.pallas import tpu as pltpu
from jax.experimental.pallas import tpu_sc as plsc

HID = 768
NCHUNK = HID // 16
ANS_NUM = 5000
OCR_NUM = 50
BATCH = 32
DEC_LEN = 100
PAD_LEN = 112
ROWS_PER_GATHER = 16


def _rsqrt(x):
    xi = lax.bitcast_convert_type(x, jnp.int32)
    yi = jnp.int32(0x5F3759DF) - lax.shift_right_arithmetic(xi, 1)
    y = lax.bitcast_convert_type(yi, jnp.float32)
    for _ in range(3):
        y = y * (1.5 - 0.5 * x * y * y)
    return y


_GATHER_DNUMS = lax.GatherDimensionNumbers(
    offset_dims=(), collapsed_slice_dims=(0,), start_index_map=(0,))


def _permute(v, idx):
    return lax.gather(v, idx[:, None], _GATHER_DNUMS, slice_sizes=(1,),
                      mode=lax.GatherScatterMode.PROMISE_IN_BOUNDS)


def _lane_total(v):
    i = lax.iota(jnp.int32, 16)
    for st in (1, 2, 4, 8):
        v = v + _permute(v, i ^ st)
    return v


def _row_stats(read):
    zero = jnp.zeros((16,), jnp.float32)
    s = [zero] * 4
    s2 = [zero] * 4
    for j in range(NCHUNK):
        x = read(j)
        k = j % 4
        s[k] = s[k] + x
        s2[k] = s2[k] + x * x
    mu = _lane_total((s[0] + s[1]) + (s[2] + s[3])) * (1.0 / HID)
    var = _lane_total((s2[0] + s2[1]) + (s2[2] + s2[3])) * (1.0 / HID) - mu * mu
    return mu, _rsqrt(var + 1e-12)


def _sc_body(ans_hbm, ocr_hbm, prev_hbm, tt_hbm,
             ans_w_hbm, ans_b_hbm, ocr_w_hbm, ocr_b_hbm,
             emb_w_hbm, emb_b_hbm, out_hbm,
             idx_v, aidx_v, oidx_v, ans_rows0, ocr_rows0, ans_rows1,
             ocr_rows1, stage, tt_v, pw0, pdw, pb0, pdb, ew, eb, sem0, sem1):
    nc = 2
    wid = lax.axis_index("s") * nc + lax.axis_index("c")

    pltpu.sync_copy(prev_hbm.at[pl.ds(wid * PAD_LEN, PAD_LEN)], idx_v)

    ocr_base = wid * OCR_NUM
    for k in range(PAD_LEN // 16):
        v = idx_v[pl.ds(k * 16, 16)]
        t = v >= ANS_NUM
        aidx_v[pl.ds(k * 16, 16)] = jnp.where(t, 0, v)
        oidx_v[pl.ds(k * 16, 16)] = jnp.where(t, v - ANS_NUM + ocr_base, ocr_base)

    pltpu.sync_copy(ans_w_hbm, pw0)
    pltpu.sync_copy(ocr_w_hbm, pdw)
    pltpu.sync_copy(ans_b_hbm, pb0)
    pltpu.sync_copy(ocr_b_hbm, pdb)
    pltpu.sync_copy(emb_w_hbm, ew)
    pltpu.sync_copy(emb_b_hbm, eb)
    pltpu.sync_copy(tt_hbm.at[pl.ds(0, 2 * HID)], tt_v)

    mu0, rs0 = _row_stats(lambda j: tt_v[pl.ds(j * 16, 16)])
    mu1, rs1 = _row_stats(lambda j: tt_v[pl.ds(HID + j * 16, 16)])
    for j in range(NCHUNK):
        sl = pl.ds(j * 16, 16)
        pdw[sl] = pdw[sl] - pw0[sl]
        tte0 = (tt_v[sl] - mu0) * rs0 * ew[sl] + eb[sl]
        pb0[sl] = pb0[sl] + tte0
        tte1 = (tt_v[pl.ds(HID + j * 16, 16)] - mu1) * rs1 * ew[sl] + eb[sl]
        pdb[sl] = pdb[sl] + tte1 - pb0[sl]

    out_base = wid * DEC_LEN
    slots = ((ans_rows0, ocr_rows0, sem0), (ans_rows1, ocr_rows1, sem1))
    nchunks = PAD_LEN // ROWS_PER_GATHER

    def issue(c):
        a, o, sem = slots[c % 2]
        sl = pl.ds(c * ROWS_PER_GATHER, ROWS_PER_GATHER)
        ca = pltpu.async_copy(ans_hbm.at[aidx_v.at[sl]], a, sem)
        co = pltpu.async_copy(ocr_hbm.at[oidx_v.at[sl]], o, sem)
        return ca, co

    pending = {0: issue(0)}
    for c in range(nchunks):
        nrows = min(ROWS_PER_GATHER, DEC_LEN - c * ROWS_PER_GATHER)
        if c + 1 < nchunks:
            pending[c + 1] = issue(c + 1)
        ca, co = pending.pop(c)
        ca.wait()
        co.wait()
        if nrows <= 0:
            continue
        ans_rows, ocr_rows, _ = slots[c % 2]

        def row_body(r, carry, c=c, ans_rows=ans_rows, ocr_rows=ocr_rows):
            rg = c * ROWS_PER_GATHER + r
            idx_splat = plsc.load_gather(idx_v, [jnp.broadcast_to(rg, (16,))])
            t = idx_splat >= ANS_NUM
            tf = jnp.where(t, 1.0, 0.0).astype(jnp.float32)

            def read_x(j):
                xa = ans_rows[r, pl.ds(j * 16, 16)]
                xo = ocr_rows[r, pl.ds(j * 16, 16)]
                return jnp.where(t, xo, xa)

            mu_v, rs_v = _row_stats(read_x)

            for j in range(NCHUNK):
                sl = pl.ds(j * 16, 16)
                x = read_x(j)
                ws = pw0[sl] + tf * pdw[sl]
                a = ws * rs_v
                cc = (pb0[sl] + tf * pdb[sl]) - mu_v * a
                stage[pl.ds(r * HID + j * 16, 16)] = x * a + cc
            return carry

        lax.fori_loop(0, nrows, row_body, 0)
        pltpu.sync_copy(
            stage.at[pl.ds(0, nrows * HID)],
            out_hbm.at[pl.ds((out_base + c * ROWS_PER_GATHER) * HID, nrows * HID)])


def kernel(ans_emb, ocr_emb, prev_inds, ans_w, ans_b, ocr_w, ocr_b, emb_w, emb_b, tt_table):
    batch = ocr_emb.shape[0]
    ocr_flat = ocr_emb.reshape(batch * OCR_NUM, HID)
    prev_pad = jnp.pad(prev_inds, ((0, 0), (0, PAD_LEN - DEC_LEN))).reshape(-1)
    tt_flat = tt_table.reshape(-1)

    mesh = plsc.VectorSubcoreMesh(core_axis_name="c", subcore_axis_name="s")
    run = functools.partial(
        pl.kernel,
        mesh=mesh,
        compiler_params=pltpu.CompilerParams(needs_layout_passes=False),
        out_type=jax.ShapeDtypeStruct((batch * DEC_LEN * HID,), jnp.float32),
        scratch_types=[
            pltpu.VMEM((PAD_LEN,), jnp.int32),
            pltpu.VMEM((PAD_LEN,), jnp.int32),
            pltpu.VMEM((PAD_LEN,), jnp.int32),
            pltpu.VMEM((ROWS_PER_GATHER, HID), jnp.float32),
            pltpu.VMEM((ROWS_PER_GATHER, HID), jnp.float32),
            pltpu.VMEM((ROWS_PER_GATHER, HID), jnp.float32),
            pltpu.VMEM((ROWS_PER_GATHER, HID), jnp.float32),
            pltpu.VMEM((ROWS_PER_GATHER * HID,), jnp.float32),
            pltpu.VMEM((2 * HID,), jnp.float32),
            pltpu.VMEM((HID,), jnp.float32),
            pltpu.VMEM((HID,), jnp.float32),
            pltpu.VMEM((HID,), jnp.float32),
            pltpu.VMEM((HID,), jnp.float32),
            pltpu.VMEM((HID,), jnp.float32),
            pltpu.VMEM((HID,), jnp.float32),
            pltpu.SemaphoreType.DMA,
            pltpu.SemaphoreType.DMA,
        ],
    )(_sc_body)
    out = run(ans_emb, ocr_flat, prev_pad, tt_flat,
              ans_w, ans_b, ocr_w, ocr_b, emb_w, emb_b)
    return out.reshape(batch, DEC_LEN, HID)


# speedup vs baseline: 4.1955x; 1.0089x over previous
import functools

import jax
import jax.numpy as jnp
from jax import lax
from jax.experimental import pallas as pl
from jax.experimental.pallas import tpu as pltpu
from jax.experimental.pallas import tpu_sc as plsc

HID = 768
NCHUNK = HID // 16
ANS_NUM = 5000
OCR_NUM = 50
BATCH = 32
DEC_LEN = 100
RPC = 16
NCH = 7


def _rsqrt(x):
    xi = lax.bitcast_convert_type(x, jnp.int32)
    yi = jnp.int32(0x5F3759DF) - lax.shift_right_arithmetic(xi, 1)
    y = lax.bitcast_convert_type(yi, jnp.float32)
    for _ in range(3):
        y = y * (1.5 - 0.5 * x * y * y)
    return y


_GATHER_DNUMS = lax.GatherDimensionNumbers(
    offset_dims=(), collapsed_slice_dims=(0,), start_index_map=(0,))


def _permute(v, idx):
    return lax.gather(v, idx[:, None], _GATHER_DNUMS, slice_sizes=(1,),
                      mode=lax.GatherScatterMode.PROMISE_IN_BOUNDS)


def _lane_total(v):
    i = lax.iota(jnp.int32, 16)
    for st in (1, 2, 4, 8):
        v = v + _permute(v, i ^ st)
    return v


def _row_stats(read):
    zero = jnp.zeros((16,), jnp.float32)
    s = [zero] * 4
    s2 = [zero] * 4
    for j in range(NCHUNK):
        x = read(j)
        k = j % 4
        s[k] = s[k] + x
        s2[k] = s2[k] + x * x
    mu = _lane_total((s[0] + s[1]) + (s[2] + s[3])) * (1.0 / HID)
    var = _lane_total((s2[0] + s2[1]) + (s2[2] + s2[3])) * (1.0 / HID) - mu * mu
    return mu, _rsqrt(var + 1e-12)


def _sc_body(ans_hbm, ocr_hbm, prev_hbm, tt_hbm,
             ans_w_hbm, ans_b_hbm, ocr_w_hbm, ocr_b_hbm,
             emb_w_hbm, emb_b_hbm, out_hbm,
             idx_v, aidx_v, oidx_v, rows_v, stage, tt_v, pwt, pbt,
             ew, eb, sem0, sem1):
    nc = 2
    wid = lax.axis_index("s") * nc + lax.axis_index("c")
    iota = lax.iota(jnp.int32, 16)
    zeros_i = jnp.zeros((16,), jnp.int32)

    tok0 = wid * DEC_LEN
    r8 = lax.bitwise_and(tok0, 7)
    idx_v[pl.ds(96, 16)] = zeros_i
    idx_v[pl.ds(112, 16)] = zeros_i
    abase = pl.multiple_of(tok0 - r8, 8)
    pltpu.sync_copy(prev_hbm.at[pl.ds(abase, 104)], idx_v.at[pl.ds(0, 104)])

    ocr_base = wid * OCR_NUM
    for k in range(NCH):
        v = plsc.load_gather(idx_v, [jnp.broadcast_to(r8 + k * 16, (16,)) + iota])
        t = v >= ANS_NUM
        aidx_v[pl.ds(k * 16, 16)] = jnp.where(t, 0, v)
        oidx_v[pl.ds(k * 16, 16)] = jnp.where(t, v - ANS_NUM + ocr_base, ocr_base)

    pltpu.sync_copy(ans_w_hbm, pwt.at[0])
    pltpu.sync_copy(ocr_w_hbm, pwt.at[1])
    pltpu.sync_copy(ans_b_hbm, pbt.at[0])
    pltpu.sync_copy(ocr_b_hbm, pbt.at[1])
    pltpu.sync_copy(emb_w_hbm, ew)
    pltpu.sync_copy(emb_b_hbm, eb)
    pltpu.sync_copy(tt_hbm.at[pl.ds(0, 2 * HID)], tt_v)

    mu0, rs0 = _row_stats(lambda j: tt_v[pl.ds(j * 16, 16)])
    mu1, rs1 = _row_stats(lambda j: tt_v[pl.ds(HID + j * 16, 16)])
    for j in range(NCHUNK):
        sl = pl.ds(j * 16, 16)
        tte0 = (tt_v[sl] - mu0) * rs0 * ew[sl] + eb[sl]
        pbt[0, sl] = pbt[0, sl] + tte0
        tte1 = (tt_v[pl.ds(HID + j * 16, 16)] - mu1) * rs1 * ew[sl] + eb[sl]
        pbt[1, sl] = pbt[1, sl] + tte1

    out_base = wid * DEC_LEN

    def issue(ck, slot):
        sem = sem0 if slot == 0 else sem1
        sl = pl.ds(ck * RPC, RPC)
        pltpu.async_copy(ans_hbm.at[aidx_v.at[sl]],
                         rows_v.at[pl.ds(slot * 32, RPC)], sem)
        pltpu.async_copy(ocr_hbm.at[oidx_v.at[sl]],
                         rows_v.at[pl.ds(slot * 32 + RPC, RPC)], sem)

    def drain(slot):
        sem = sem0 if slot == 0 else sem1
        pltpu.make_async_copy(ans_hbm.at[aidx_v.at[pl.ds(0, RPC)]],
                              rows_v.at[pl.ds(slot * 32, RPC)], sem).wait()
        pltpu.make_async_copy(ocr_hbm.at[oidx_v.at[pl.ds(0, RPC)]],
                              rows_v.at[pl.ds(slot * 32 + RPC, RPC)], sem).wait()

    def compute_chunk(ck, slot, nrows):
        def row_body(r, carry):
            tok = r8 + ck * RPC + r
            idx_splat = plsc.load_gather(idx_v, [jnp.broadcast_to(tok, (16,))])
            t_i32 = jnp.where(idx_splat >= ANS_NUM, 1, 0)
            xrow = jnp.broadcast_to(slot * 32 + r, (16,)) + t_i32 * RPC

            cols = [iota + (j * 16) for j in range(NCHUNK)]
            mu, rs = _row_stats(
                lambda j: plsc.load_gather(rows_v, [xrow, cols[j]]))

            for j in range(NCHUNK):
                x = plsc.load_gather(rows_v, [xrow, cols[j]])
                wv = plsc.load_gather(pwt, [t_i32, cols[j]])
                bv = plsc.load_gather(pbt, [t_i32, cols[j]])
                a = wv * rs
                cc = bv - mu * a
                stage[pl.ds(r * HID + j * 16, 16)] = x * a + cc
            return carry

        lax.fori_loop(0, nrows, row_body, 0)
        pltpu.sync_copy(
            stage.at[pl.ds(0, nrows * HID)],
            out_hbm.at[pl.ds((out_base + ck * RPC) * HID, nrows * HID)])

    issue(0, 0)
    issue(1, 1)

    def pair_body(k, carry):
        c0 = 2 * k
        drain(0)
        compute_chunk(c0, 0, RPC)
        issue(jnp.minimum(c0 + 2, NCH - 1), 0)
        drain(1)
        compute_chunk(c0 + 1, 1, RPC)
        issue(jnp.minimum(c0 + 3, NCH - 1), 1)
        return carry

    lax.fori_loop(0, 3, pair_body, 0)
    drain(0)
    drain(1)
    compute_chunk(NCH - 1, 0, DEC_LEN - (NCH - 1) * RPC)


def kernel(ans_emb, ocr_emb, prev_inds, ans_w, ans_b, ocr_w, ocr_b, emb_w, emb_b, tt_table):
    batch = ocr_emb.shape[0]
    ocr_flat = ocr_emb.reshape(batch * OCR_NUM, HID)
    prev_flat = prev_inds.reshape(-1)
    tt_flat = tt_table.reshape(-1)

    mesh = plsc.VectorSubcoreMesh(core_axis_name="c", subcore_axis_name="s")
    run = functools.partial(
        pl.kernel,
        mesh=mesh,
        compiler_params=pltpu.CompilerParams(needs_layout_passes=False),
        out_type=jax.ShapeDtypeStruct((batch * DEC_LEN * HID,), jnp.float32),
        scratch_types=[
            pltpu.VMEM((128,), jnp.int32),
            pltpu.VMEM((NCH * RPC,), jnp.int32),
            pltpu.VMEM((NCH * RPC,), jnp.int32),
            pltpu.VMEM((64, HID), jnp.float32),
            pltpu.VMEM((RPC * HID,), jnp.float32),
            pltpu.VMEM((2 * HID,), jnp.float32),
            pltpu.VMEM((2, HID), jnp.float32),
            pltpu.VMEM((2, HID), jnp.float32),
            pltpu.VMEM((HID,), jnp.float32),
            pltpu.VMEM((HID,), jnp.float32),
            pltpu.SemaphoreType.DMA,
            pltpu.SemaphoreType.DMA,
        ],
    )(_sc_body)
    out = run(ans_emb, ocr_flat, prev_flat, tt_flat,
              ans_w, ans_b, ocr_w, ocr_b, emb_w, emb_b)
    return out.reshape(batch, DEC_LEN, HID)


# speedup vs baseline: 5.2198x; 1.2442x over previous
import functools

import jax
import jax.numpy as jnp
from jax import lax
from jax.experimental import pallas as pl
from jax.experimental.pallas import tpu as pltpu
from jax.experimental.pallas import tpu_sc as plsc

HID = 768
NCHUNK = HID // 16
ANS_NUM = 5000
OCR_NUM = 50
BATCH = 32
DEC_LEN = 100
RPC = 16
NCH = 7


def _rsqrt(x):
    xi = lax.bitcast_convert_type(x, jnp.int32)
    yi = jnp.int32(0x5F3759DF) - lax.shift_right_arithmetic(xi, 1)
    y = lax.bitcast_convert_type(yi, jnp.float32)
    for _ in range(3):
        y = y * (1.5 - 0.5 * x * y * y)
    return y


_GATHER_DNUMS = lax.GatherDimensionNumbers(
    offset_dims=(), collapsed_slice_dims=(0,), start_index_map=(0,))


def _permute(v, idx):
    return lax.gather(v, idx[:, None], _GATHER_DNUMS, slice_sizes=(1,),
                      mode=lax.GatherScatterMode.PROMISE_IN_BOUNDS)


def _lane_total(v):
    i = lax.iota(jnp.int32, 16)
    for st in (1, 2, 4, 8):
        v = v + _permute(v, i ^ st)
    return v


def _row_stats(read):
    zero = jnp.zeros((16,), jnp.float32)
    s = [zero] * 4
    s2 = [zero] * 4
    for j in range(NCHUNK):
        x = read(j)
        k = j % 4
        s[k] = s[k] + x
        s2[k] = s2[k] + x * x
    mu = _lane_total((s[0] + s[1]) + (s[2] + s[3])) * (1.0 / HID)
    var = _lane_total((s2[0] + s2[1]) + (s2[2] + s2[3])) * (1.0 / HID) - mu * mu
    return mu, _rsqrt(var + 1e-12)


def _sc_body(ans_hbm, ocr_hbm, prev_hbm, tt_hbm,
             ans_w_hbm, ans_b_hbm, ocr_w_hbm, ocr_b_hbm,
             emb_w_hbm, emb_b_hbm, out_hbm,
             idx_v, aidx_v, oidx_v, didx_v, rows_v, stage, tt_v, pwt, pbt,
             ew, eb, sem0, sem1):
    nc = 2
    wid = lax.axis_index("s") * nc + lax.axis_index("c")
    iota = lax.iota(jnp.int32, 16)
    zeros_i = jnp.zeros((16,), jnp.int32)

    tok0 = wid * DEC_LEN
    r8 = lax.bitwise_and(tok0, 7)
    idx_v[pl.ds(96, 16)] = zeros_i
    idx_v[pl.ds(112, 16)] = zeros_i
    abase = pl.multiple_of(tok0 - r8, 8)
    pltpu.sync_copy(prev_hbm.at[pl.ds(abase, 104)], idx_v.at[pl.ds(0, 104)])

    for k in range(NCH):
        v = plsc.load_gather(idx_v, [jnp.broadcast_to(r8 + k * 16, (16,)) + iota])
        t = v >= ANS_NUM
        aidx_v[pl.ds(k * 16, 16)] = jnp.where(t, 0, v)
        oidx_v[pl.ds(k * 16, 16)] = jnp.where(t, (v - ANS_NUM) * BATCH + wid, wid)
        tokv = iota + (k * 16)
        if k == NCH - 1:
            tokv = jnp.where(iota < 4, tokv, iota - 4)
        didx_v[k, :] = tokv * BATCH + wid

    pltpu.sync_copy(ans_w_hbm, pwt.at[0])
    pltpu.sync_copy(ocr_w_hbm, pwt.at[1])
    pltpu.sync_copy(ans_b_hbm, pbt.at[0])
    pltpu.sync_copy(ocr_b_hbm, pbt.at[1])
    pltpu.sync_copy(emb_w_hbm, ew)
    pltpu.sync_copy(emb_b_hbm, eb)
    pltpu.sync_copy(tt_hbm.at[pl.ds(0, 2 * HID)], tt_v)

    mu0, rs0 = _row_stats(lambda j: tt_v[pl.ds(j * 16, 16)])
    mu1, rs1 = _row_stats(lambda j: tt_v[pl.ds(HID + j * 16, 16)])
    for j in range(NCHUNK):
        sl = pl.ds(j * 16, 16)
        tte0 = (tt_v[sl] - mu0) * rs0 * ew[sl] + eb[sl]
        pbt[0, sl] = pbt[0, sl] + tte0
        tte1 = (tt_v[pl.ds(HID + j * 16, 16)] - mu1) * rs1 * ew[sl] + eb[sl]
        pbt[1, sl] = pbt[1, sl] + tte1

    def issue(ck, slot):
        sem = sem0 if slot == 0 else sem1
        sl = pl.ds(ck * RPC, RPC)
        ca = pltpu.async_copy(ans_hbm.at[aidx_v.at[sl]],
                              rows_v.at[pl.ds(slot * 32, RPC)], sem)
        co = pltpu.async_copy(ocr_hbm.at[oidx_v.at[sl]],
                              rows_v.at[pl.ds(slot * 32 + RPC, RPC)], sem)
        return ca, co

    def compute_chunk(ck, slot):
        def row_body(r, carry):
            tok = r8 + ck * RPC + r
            idx_splat = plsc.load_gather(idx_v, [jnp.broadcast_to(tok, (16,))])
            t_i32 = jnp.where(idx_splat >= ANS_NUM, 1, 0)
            xrow = jnp.broadcast_to(slot * 32 + r, (16,)) + t_i32 * RPC

            cols = [iota + (j * 16) for j in range(NCHUNK)]
            mu, rs = _row_stats(
                lambda j: plsc.load_gather(rows_v, [xrow, cols[j]]))

            for j in range(NCHUNK):
                x = plsc.load_gather(rows_v, [xrow, cols[j]])
                wv = plsc.load_gather(pwt, [t_i32, cols[j]])
                bv = plsc.load_gather(pbt, [t_i32, cols[j]])
                a = wv * rs
                cc = bv - mu * a
                stage[r, pl.ds(j * 16, 16)] = x * a + cc
            return carry

        lax.fori_loop(0, RPC, row_body, 0)
        pltpu.sync_copy(stage, out_hbm.at[didx_v.at[ck]])

    order = [NCH - 1] + list(range(NCH - 1))
    pending = {0: issue(order[0], 0), 1: issue(order[1], 1)}
    for i, ck in enumerate(order):
        slot = i % 2
        ca, co = pending.pop(i)
        ca.wait()
        co.wait()
        compute_chunk(ck, slot)
        if i + 2 < NCH:
            pending[i + 2] = issue(order[i + 2], slot)


def kernel(ans_emb, ocr_emb, prev_inds, ans_w, ans_b, ocr_w, ocr_b, emb_w, emb_b, tt_table):
    batch = ocr_emb.shape[0]
    ocr_flat = jnp.transpose(ocr_emb, (1, 0, 2)).reshape(OCR_NUM * batch, HID)
    prev_flat = prev_inds.reshape(-1)
    tt_flat = tt_table.reshape(-1)
    mesh = plsc.VectorSubcoreMesh(core_axis_name="c", subcore_axis_name="s")
    run = functools.partial(
        pl.kernel,
        mesh=mesh,
        compiler_params=pltpu.CompilerParams(needs_layout_passes=False),
        out_type=jax.ShapeDtypeStruct((DEC_LEN * batch, HID), jnp.float32),
        scratch_types=[
            pltpu.VMEM((128,), jnp.int32),
            pltpu.VMEM((NCH * RPC,), jnp.int32),
            pltpu.VMEM((NCH * RPC,), jnp.int32),
            pltpu.VMEM((NCH, RPC), jnp.int32),
            pltpu.VMEM((64, HID), jnp.float32),
            pltpu.VMEM((RPC, HID), jnp.float32),
            pltpu.VMEM((2 * HID,), jnp.float32),
            pltpu.VMEM((2, HID), jnp.float32),
            pltpu.VMEM((2, HID), jnp.float32),
            pltpu.VMEM((HID,), jnp.float32),
            pltpu.VMEM((HID,), jnp.float32),
            pltpu.SemaphoreType.DMA,
            pltpu.SemaphoreType.DMA,
        ],
    )(_sc_body)
    out = run(ans_emb, ocr_flat, prev_flat, tt_flat,
              ans_w, ans_b, ocr_w, ocr_b, emb_w, emb_b)
    return jnp.transpose(out.reshape(DEC_LEN, batch, HID), (1, 0, 2))
